# Initial kernel scaffold; baseline (speedup 1.0000x reference)
import jax
import jax.numpy as jnp
from jax.experimental import pallas as pl


def kernel(x, edge_index, edge_attr, layer_emb, color_emb, relsize_emb, W1, b1, W2, b2, W3, b3, Wp, bp):
    raise NotImplementedError("write your pallas kernel here")


# speedup vs baseline: 3.2716x; 3.2716x over previous
import functools

import jax
import jax.numpy as jnp
from jax import lax
from jax.experimental import pallas as pl
from jax.experimental.pallas import tpu as pltpu
from jax.experimental.pallas import tpu_sc as plsc

N = 10000
E = 160000
FS = 1000
NC, NS = 2, 16
NW = NC * NS
NP = 10240
BN = 1000
LRELU = 0.01


def _wrap_clip(idx, n):
  idx = jnp.where(idx < 0, idx + n, idx)
  return jnp.clip(idx, 0, n - 1)


_EPD = E // NW
_DCH = 1000


@functools.partial(
    pl.kernel,
    out_type=jax.ShapeDtypeStruct((NW, N), jnp.float32),
    mesh=plsc.VectorSubcoreMesh(core_axis_name="c", subcore_axis_name="s"),
    scratch_types=[
        pltpu.VMEM((N,), jnp.float32),
        pltpu.VMEM((_DCH + 16,), jnp.int32),
        pltpu.VMEM((_DCH + 16,), jnp.float32),
    ],
    compiler_params=pltpu.CompilerParams(needs_layout_passes=False),
    name="sc_deg",
)
def _sc_deg(dst_hbm, ew_hbm, out_hbm, deg_v, idx_v, w_v):
  c = lax.axis_index("c")
  s = lax.axis_index("s")
  wid = s * NC + c

  def zero(i, _):
    deg_v[pl.ds(i * 16, 16)] = jnp.zeros((16,), jnp.float32)
    return 0

  lax.fori_loop(0, N // 16, zero, 0)
  idx_v[pl.ds(_DCH, 16)] = jnp.zeros((16,), jnp.int32)
  w_v[pl.ds(_DCH, 16)] = jnp.zeros((16,), jnp.float32)
  base0 = wid * _EPD
  ngroups = (_DCH + 15) // 16

  def chunk(ci, _):
    b = pl.multiple_of(base0 + ci * _DCH, 8)
    pltpu.sync_copy(dst_hbm.at[pl.ds(b, _DCH)], idx_v.at[pl.ds(0, _DCH)])
    pltpu.sync_copy(ew_hbm.at[pl.ds(b, _DCH)], w_v.at[pl.ds(0, _DCH)])

    def acc(g, _):
      sl = pl.ds(g * 16, 16)
      plsc.addupdate_scatter(deg_v, [idx_v[sl]], w_v[sl])
      return 0

    lax.fori_loop(0, ngroups, acc, 0)
    return 0

  lax.fori_loop(0, _EPD // _DCH, chunk, 0)
  pltpu.sync_copy(deg_v, out_hbm.at[wid])


def _make_sc_edge(K, name):
  Fc = 128
  CPS = max(1, K // NC)
  EP = E // NS if K >= 2 else E // NW
  B = 80 if K >= 2 else 40
  BP = B if B % 16 == 0 else B + 16 - B % 16
  NB = EP // B
  RPT = NP // NS
  ZR = 128
  n_out = K if K >= 2 else NC

  @functools.partial(
      pl.kernel,
      out_type=jax.ShapeDtypeStruct((n_out, NP, Fc), jnp.float32),
      mesh=plsc.VectorSubcoreMesh(core_axis_name="c", subcore_axis_name="s"),
      scratch_types=[
          pltpu.VMEM((N,), jnp.float32),
          pltpu.VMEM((BP,), jnp.int32),
          pltpu.VMEM((BP,), jnp.int32),
          pltpu.VMEM((BP,), jnp.float32),
          pltpu.VMEM((BP,), jnp.float32),
          pltpu.VMEM((BP, Fc), jnp.float32),
          pltpu.VMEM((ZR, Fc), jnp.float32),
          pltpu.VMEM_SHARED((NP, Fc), jnp.float32),
          pltpu.SemaphoreType.DMA,
      ],
      compiler_params=pltpu.CompilerParams(needs_layout_passes=False),
      name=name,
  )
  def edge(hw_hbm, dinv_hbm, src_hbm, dst_hbm, ew_hbm, out_hbm,
           dinv_v, si_v, di_v, ew_v, f_v, rows_v, zero_v, agg_sh, sem):
    c = lax.axis_index("c")
    s = lax.axis_index("s")
    pltpu.sync_copy(dinv_hbm, dinv_v)

    def zz(i, _):
      for j in range(Fc // 16):
        zero_v[i, pl.ds(j * 16, 16)] = jnp.zeros((16,), jnp.float32)
      return 0

    lax.fori_loop(0, ZR, zz, 0)
    if BP != B:
      pad = pl.ds(BP - 16, 16)
      si_v[pad] = jnp.zeros((16,), jnp.int32)
      di_v[pad] = jnp.zeros((16,), jnp.int32)
      ew_v[pad] = jnp.zeros((16,), jnp.float32)
    if K >= 2:
      edge_base = s * EP
    else:
      edge_base = c * (E // NC) + s * EP

    for cc in range(CPS):
      kk = c + cc * NC if K >= 2 else 0
      for z in range(RPT // ZR):
        pltpu.sync_copy(zero_v, agg_sh.at[pl.ds(s * RPT + z * ZR, ZR)])
      plsc.subcore_barrier()

      def batch(bi, _):
        b = pl.multiple_of(edge_base + bi * B, 8)
        pltpu.sync_copy(src_hbm.at[pl.ds(b, B)], si_v.at[pl.ds(0, B)])
        pltpu.sync_copy(dst_hbm.at[pl.ds(b, B)], di_v.at[pl.ds(0, B)])
        pltpu.sync_copy(ew_hbm.at[pl.ds(b, B)], ew_v.at[pl.ds(0, B)])
        for g in range(BP // 16):
          sl = pl.ds(g * 16, 16)
          sidx = si_v[sl]
          dv_s = plsc.load_gather(dinv_v, [sidx])
          dv_d = plsc.load_gather(dinv_v, [di_v[sl]])
          f_v[sl] = dv_s * dv_d * ew_v[sl]
          if K >= 2:
            si_v[sl] = sidx + kk * N

        pltpu.async_copy(hw_hbm.at[si_v], rows_v, sem).wait()

        for g in range(BP // 16):
          fvec = f_v[pl.ds(g * 16, 16)]
          for lane in range(16):
            i = g * 16 + lane
            fs = fvec[lane]
            for j in range(Fc // 16):
              sl2 = pl.ds(j * 16, 16)
              rows_v[i, sl2] = rows_v[i, sl2] * fs

        pltpu.sync_copy(rows_v, agg_sh.at[di_v], add=True)
        return 0

      lax.fori_loop(0, NB, batch, 0)
      plsc.subcore_barrier()
      out_idx = kk if K >= 2 else c
      pltpu.sync_copy(agg_sh.at[pl.ds(s * RPT, RPT)],
                      out_hbm.at[out_idx, pl.ds(s * RPT, RPT)])
      if cc + 1 < CPS:
        plsc.subcore_barrier()

  return edge


_sc_edge1 = _make_sc_edge(4, "sc_edge1")
_sc_edge2 = _make_sc_edge(2, "sc_edge2")
_sc_edge3 = _make_sc_edge(1, "sc_edge3")


def _tc_dinv_body(dp_ref, out_ref):
  deg = jnp.sum(dp_ref[...], axis=1, keepdims=True) + 1.0
  out_ref[...] = jnp.where(deg > 0, lax.rsqrt(deg), 0.0)


def _tc_prep_body(x_ref, lemb_ref, remb_ref, cemb_ref, w1_ref, out_ref):
  xb = x_ref[...]
  lid = _wrap_clip((xb[:, 0:1] - 1.0).astype(jnp.int32), 3)
  rid = _wrap_clip(
      jnp.round(jnp.abs(xb[:, FS + 1:FS + 2]) * 10.0).astype(jnp.int32), 11)
  resnet = xb[:, 1:1 + FS]

  hw = jnp.dot(resnet, w1_ref[250:250 + FS, :],
               preferred_element_type=jnp.float32)
  for t in range(3):
    row = jnp.dot(lemb_ref[t:t + 1, :], w1_ref[0:250, :],
                  preferred_element_type=jnp.float32)
    hw = hw + jnp.where(lid == t, 1.0, 0.0) * row
  for t in range(11):
    row = jnp.dot(remb_ref[t:t + 1, :], w1_ref[1250:1500, :],
                  preferred_element_type=jnp.float32)
    hw = hw + jnp.where(rid == t, 1.0, 0.0) * row
  iot = lax.broadcasted_iota(jnp.int32, (BN, 256), 1)
  for k in range(3):
    cid = _wrap_clip(xb[:, FS + 2 + k:FS + 3 + k].astype(jnp.int32), 256)
    oh = (iot == cid).astype(jnp.float32)
    col = jnp.dot(oh, cemb_ref[...], preferred_element_type=jnp.float32)
    hw = hw + jnp.dot(col, w1_ref[1500 + 85 * k:1585 + 85 * k, :],
                      preferred_element_type=jnp.float32)
  for k in range(4):
    out_ref[k] = hw[:, 128 * k:128 * (k + 1)]


def _make_tc_mid(K_in, Fc_in, K_out, Fc_out):
  def body(agg_ref, hw_ref, dinv_ref, b_ref, w_ref, out_ref):
    aggb = jnp.concatenate([agg_ref[k] for k in range(K_in)], axis=1)
    hwb = jnp.concatenate([hw_ref[k] for k in range(K_in)], axis=1)
    dv = dinv_ref[...]
    pre = aggb + dv * dv * hwb + b_ref[...]
    h = jnp.where(pre >= 0, pre, LRELU * pre)
    hw = jnp.dot(h, w_ref[...], preferred_element_type=jnp.float32)
    for k in range(K_out):
      out_ref[k] = hw[:, Fc_out * k:Fc_out * (k + 1)]
  return body


_tc_mid2_body = _make_tc_mid(4, 128, 2, 128)


def _tc_mid3_body(agg_ref, hw_ref, dinv_ref, b_ref, w_ref, out_ref):
  aggb = jnp.concatenate([agg_ref[k] for k in range(2)], axis=1)
  hwb = jnp.concatenate([hw_ref[k] for k in range(2)], axis=1)
  dv = dinv_ref[...]
  pre = aggb + dv * dv * hwb + b_ref[...]
  h = jnp.where(pre >= 0, pre, LRELU * pre)
  hw = jnp.dot(h, w_ref[...], preferred_element_type=jnp.float32)
  out_ref[...] = jnp.concatenate(
      [hw, jnp.zeros((BN, 64), jnp.float32)], axis=1)


def _tc_final_body(agg_ref, hw_ref, dinv_ref, b_ref, wp_ref, bp_ref, out_ref):
  aggb = (agg_ref[0] + agg_ref[1])[:, 0:64]
  hwb = hw_ref[...][:, 0:64]
  dv = dinv_ref[...]
  pre = aggb + dv * dv * hwb + b_ref[...]
  h = jnp.where(pre >= 0, pre, LRELU * pre)
  out_ref[...] = jnp.dot(h, wp_ref[...],
                         preferred_element_type=jnp.float32) + bp_ref[...]


def _row_blk(shape):
  return pl.BlockSpec((BN,) + shape[1:], lambda i: (i,) + (0,) * (len(shape) - 1))


def _chunk_blk(K, Fc):
  return pl.BlockSpec((K, BN, Fc), lambda i: (0, i, 0))


def _full_blk(shape):
  return pl.BlockSpec(shape, lambda i: (0,) * len(shape))


def kernel(x, edge_index, edge_attr, layer_emb, color_emb, relsize_emb,
           W1, b1, W2, b2, W3, b3, Wp, bp):
  src = edge_index[0]
  dst = edge_index[1]
  grid = (N // BN,)

  deg_parts = _sc_deg(dst, edge_attr)
  dinv = pl.pallas_call(
      _tc_dinv_body,
      grid=grid,
      in_specs=[_row_blk((N, NW))],
      out_specs=_row_blk((N, 1)),
      out_shape=jax.ShapeDtypeStruct((N, 1), jnp.float32),
  )(deg_parts.T)
  dinv_flat = dinv.reshape(N)

  hw1 = pl.pallas_call(
      _tc_prep_body,
      grid=grid,
      in_specs=[
          _row_blk(x.shape),
          _full_blk((3, 250)),
          _full_blk((11, 250)),
          _full_blk((256, 85)),
          _full_blk((250 + FS + 250 + 255, 512)),
      ],
      out_specs=_chunk_blk(4, 128),
      out_shape=jax.ShapeDtypeStruct((4, N, 128), jnp.float32),
  )(x, layer_emb, relsize_emb, color_emb, W1)

  agg1 = _sc_edge1(hw1.reshape(4 * N, 128), dinv_flat, src, dst, edge_attr)

  hw2 = pl.pallas_call(
      _tc_mid2_body,
      grid=grid,
      in_specs=[
          _chunk_blk(4, 128),
          _chunk_blk(4, 128),
          _row_blk((N, 1)),
          _full_blk((1, 512)),
          _full_blk((512, 256)),
      ],
      out_specs=_chunk_blk(2, 128),
      out_shape=jax.ShapeDtypeStruct((2, N, 128), jnp.float32),
  )(agg1, hw1, dinv, b1.reshape(1, 512), W2)

  agg2 = _sc_edge2(hw2.reshape(2 * N, 128), dinv_flat, src, dst, edge_attr)

  hw3 = pl.pallas_call(
      _tc_mid3_body,
      grid=grid,
      in_specs=[
          _chunk_blk(2, 128),
          _chunk_blk(2, 128),
          _row_blk((N, 1)),
          _full_blk((1, 256)),
          _full_blk((256, 64)),
      ],
      out_specs=_row_blk((N, 128)),
      out_shape=jax.ShapeDtypeStruct((N, 128), jnp.float32),
  )(agg2, hw2, dinv, b2.reshape(1, 256), W3)

  agg3 = _sc_edge3(hw3, dinv_flat, src, dst, edge_attr)

  out = pl.pallas_call(
      _tc_final_body,
      grid=grid,
      in_specs=[
          _chunk_blk(2, 128),
          _row_blk((N, 128)),
          _row_blk((N, 1)),
          _full_blk((1, 64)),
          _full_blk((64, 3)),
          _full_blk((1, 3)),
      ],
      out_specs=_row_blk((N, 3)),
      out_shape=jax.ShapeDtypeStruct((N, 3), jnp.float32),
  )(agg3, hw3, dinv, b3.reshape(1, 64), Wp, bp.reshape(1, 3))
  return out


# speedup vs baseline: 9.8812x; 3.0203x over previous
import functools

import jax
import jax.numpy as jnp
from jax import lax
from jax.experimental import pallas as pl
from jax.experimental.pallas import tpu as pltpu
from jax.experimental.pallas import tpu_sc as plsc

N = 10000
E = 160000
FS = 1000
NC, NS = 2, 16
NW = NC * NS
NP = 10240
BN = 1000
LRELU = 0.01


def _wrap_clip(idx, n):
  idx = jnp.where(idx < 0, idx + n, idx)
  return jnp.clip(idx, 0, n - 1)


_EPD = E // NW
_DCH = 1000


@functools.partial(
    pl.kernel,
    out_type=jax.ShapeDtypeStruct((NW, N), jnp.float32),
    mesh=plsc.VectorSubcoreMesh(core_axis_name="c", subcore_axis_name="s"),
    scratch_types=[
        pltpu.VMEM((N,), jnp.float32),
        pltpu.VMEM((_DCH + 16,), jnp.int32),
        pltpu.VMEM((_DCH + 16,), jnp.float32),
    ],
    compiler_params=pltpu.CompilerParams(needs_layout_passes=False),
    name="sc_deg",
)
def _sc_deg(dst_hbm, ew_hbm, out_hbm, deg_v, idx_v, w_v):
  c = lax.axis_index("c")
  s = lax.axis_index("s")
  wid = s * NC + c

  def zero(i, _):
    deg_v[pl.ds(i * 16, 16)] = jnp.zeros((16,), jnp.float32)
    return 0

  lax.fori_loop(0, N // 16, zero, 0)
  idx_v[pl.ds(_DCH, 16)] = jnp.zeros((16,), jnp.int32)
  w_v[pl.ds(_DCH, 16)] = jnp.zeros((16,), jnp.float32)
  base0 = wid * _EPD
  ngroups = (_DCH + 15) // 16

  def chunk(ci, _):
    b = pl.multiple_of(base0 + ci * _DCH, 8)
    pltpu.sync_copy(dst_hbm.at[pl.ds(b, _DCH)], idx_v.at[pl.ds(0, _DCH)])
    pltpu.sync_copy(ew_hbm.at[pl.ds(b, _DCH)], w_v.at[pl.ds(0, _DCH)])

    def acc(g, _):
      sl = pl.ds(g * 16, 16)
      plsc.addupdate_scatter(deg_v, [idx_v[sl]], w_v[sl])
      return 0

    lax.fori_loop(0, ngroups, acc, 0)
    return 0

  lax.fori_loop(0, _EPD // _DCH, chunk, 0)
  pltpu.sync_copy(deg_v, out_hbm.at[wid])


def _make_sc_edge(K, name):
  Fc = 128
  CPS = max(1, K // NC)
  EP = E // NS if K >= 2 else E // NW
  B = 80 if K >= 2 else 40
  BP = B if B % 16 == 0 else B + 16 - B % 16
  NB = EP // B
  SB = 25
  SE = SB * B
  SP = SE if SE % 16 == 0 else SE + 16 - SE % 16
  RPT = NP // NS
  ZR = 32
  n_out = K if K >= 2 else NC

  @functools.partial(
      pl.kernel,
      out_type=jax.ShapeDtypeStruct((n_out, NP, Fc), jnp.float32),
      mesh=plsc.VectorSubcoreMesh(core_axis_name="c", subcore_axis_name="s"),
      scratch_types=[
          pltpu.VMEM((N,), jnp.float32),
          pltpu.VMEM((SP,), jnp.int32),
          pltpu.VMEM((SP,), jnp.int32),
          pltpu.VMEM((SP,), jnp.float32),
          pltpu.VMEM((BP,), jnp.int32),
          pltpu.VMEM((BP,), jnp.int32),
          pltpu.VMEM((BP,), jnp.int32),
          pltpu.VMEM((BP,), jnp.int32),
          pltpu.VMEM((BP,), jnp.float32),
          pltpu.VMEM((BP,), jnp.float32),
          pltpu.VMEM((BP, Fc), jnp.float32),
          pltpu.VMEM((BP, Fc), jnp.float32),
          pltpu.VMEM((ZR, Fc), jnp.float32),
          pltpu.VMEM_SHARED((NP, Fc), jnp.float32),
          pltpu.SemaphoreType.DMA,
          pltpu.SemaphoreType.DMA,
      ],
      compiler_params=pltpu.CompilerParams(needs_layout_passes=False),
      name=name,
  )
  def edge(hw_hbm, dinv_hbm, src_hbm, dst_hbm, ew_hbm, out_hbm,
           dinv_v, sa_v, da_v, wa_v, si0, si1, di0, di1, f0, f1,
           rows0, rows1, zero_v, agg_sh, sem0, sem1):
    c = lax.axis_index("c")
    s = lax.axis_index("s")
    pltpu.sync_copy(dinv_hbm, dinv_v)
    if SP != SE:
      sa_v[pl.ds(SP - 16, 16)] = jnp.zeros((16,), jnp.int32)
      da_v[pl.ds(SP - 16, 16)] = jnp.zeros((16,), jnp.int32)
      wa_v[pl.ds(SP - 16, 16)] = jnp.zeros((16,), jnp.float32)
    if K >= 2:
      edge_base = s * EP
    else:
      edge_base = c * (E // NC) + s * EP
    edge_base = pl.multiple_of(edge_base, 8)

    def zz(i, _):
      for j in range(Fc // 16):
        zero_v[i, pl.ds(j * 16, 16)] = jnp.zeros((16,), jnp.float32)
      return 0

    lax.fori_loop(0, ZR, zz, 0)
    iota16 = lax.iota(jnp.int32, 16)
    padmask = iota16 < (16 - (BP - B))
    padf = jnp.where(padmask, 1.0, 0.0)

    for cc in range(CPS):
      kk = c + cc * NC if K >= 2 else 0

      def fstage(bb, si_x, di_x, f_x):
        off = bb * B
        for g in range(BP // 16):
          sl = pl.ds(g * 16, 16)
          so = pl.ds(off + g * 16, 16)
          sidx = sa_v[so]
          didx = da_v[so]
          dv_s = plsc.load_gather(dinv_v, [sidx])
          dv_d = plsc.load_gather(dinv_v, [didx])
          f_x[sl] = dv_s * dv_d * wa_v[so]
          di_x[sl] = didx
          if K >= 2:
            si_x[sl] = sidx + kk * N
          else:
            si_x[sl] = sidx
        if BP != B:
          tl = pl.ds(BP - 16, 16)
          f_x[tl] = f_x[tl] * padf
          di_x[tl] = jnp.where(padmask, di_x[tl], N + s * 8 + (iota16 - 8))

      def gather(si_x, rows_x, sem_x):
        return pltpu.async_copy(hw_hbm.at[si_x], rows_x, sem_x)

      def consume(f_x, rows_x, di_x):
        for g in range(BP // 16):
          fvec = f_x[pl.ds(g * 16, 16)]
          for lane in range(16):
            i = g * 16 + lane
            fs = fvec[lane]
            for j in range(Fc // 16):
              sl2 = pl.ds(j * 16, 16)
              rows_x[i, sl2] = rows_x[i, sl2] * fs
        pltpu.sync_copy(rows_x, agg_sh.at[di_x], add=True)

      for z in range(RPT // ZR):
        pltpu.sync_copy(zero_v, agg_sh.at[pl.ds(s * RPT + z * ZR, ZR)])
      plsc.subcore_barrier()

      def stage_loop(st, _):
        sb = pl.multiple_of(edge_base + st * SE, 8)
        pltpu.sync_copy(src_hbm.at[pl.ds(sb, SE)], sa_v.at[pl.ds(0, SE)])
        pltpu.sync_copy(dst_hbm.at[pl.ds(sb, SE)], da_v.at[pl.ds(0, SE)])
        pltpu.sync_copy(ew_hbm.at[pl.ds(sb, SE)], wa_v.at[pl.ds(0, SE)])

        fstage(0, si0, di0, f0)
        gather(si0, rows0, sem0)

        def pair(it, _):
          b0 = it * 2
          fstage(b0 + 1, si1, di1, f1)
          gather(si1, rows1, sem1)
          pltpu.make_async_copy(hw_hbm.at[si0], rows0, sem0).wait()
          consume(f0, rows0, di0)
          fstage(b0 + 2, si0, di0, f0)
          gather(si0, rows0, sem0)
          pltpu.make_async_copy(hw_hbm.at[si1], rows1, sem1).wait()
          consume(f1, rows1, di1)
          return 0

        lax.fori_loop(0, (SB - 1) // 2, pair, 0)
        pltpu.make_async_copy(hw_hbm.at[si0], rows0, sem0).wait()
        consume(f0, rows0, di0)
        return 0

      lax.fori_loop(0, NB // SB, stage_loop, 0)
      plsc.subcore_barrier()
      out_idx = kk if K >= 2 else c
      pltpu.sync_copy(agg_sh.at[pl.ds(s * RPT, RPT)],
                      out_hbm.at[out_idx, pl.ds(s * RPT, RPT)])
      if cc + 1 < CPS:
        plsc.subcore_barrier()

  return edge


_sc_edge1 = _make_sc_edge(4, "sc_edge1")
_sc_edge2 = _make_sc_edge(2, "sc_edge2")
_sc_edge3 = _make_sc_edge(1, "sc_edge3")


def _tc_dinv_body(dp_ref, out_ref):
  deg = jnp.sum(dp_ref[...], axis=1, keepdims=True) + 1.0
  out_ref[...] = jnp.where(deg > 0, lax.rsqrt(deg), 0.0)


def _tc_prep_body(x_ref, lemb_ref, remb_ref, cemb_ref, w1_ref, out_ref):
  xb = x_ref[...]
  lid = _wrap_clip((xb[:, 0:1] - 1.0).astype(jnp.int32), 3)
  rid = _wrap_clip(
      jnp.round(jnp.abs(xb[:, FS + 1:FS + 2]) * 10.0).astype(jnp.int32), 11)
  resnet = xb[:, 1:1 + FS]

  hw = jnp.dot(resnet, w1_ref[250:250 + FS, :],
               preferred_element_type=jnp.float32)
  for t in range(3):
    row = jnp.dot(lemb_ref[t:t + 1, :], w1_ref[0:250, :],
                  preferred_element_type=jnp.float32)
    hw = hw + jnp.where(lid == t, 1.0, 0.0) * row
  for t in range(11):
    row = jnp.dot(remb_ref[t:t + 1, :], w1_ref[1250:1500, :],
                  preferred_element_type=jnp.float32)
    hw = hw + jnp.where(rid == t, 1.0, 0.0) * row
  iot = lax.broadcasted_iota(jnp.int32, (BN, 256), 1)
  for k in range(3):
    cid = _wrap_clip(xb[:, FS + 2 + k:FS + 3 + k].astype(jnp.int32), 256)
    oh = (iot == cid).astype(jnp.float32)
    col = jnp.dot(oh, cemb_ref[...], preferred_element_type=jnp.float32)
    hw = hw + jnp.dot(col, w1_ref[1500 + 85 * k:1585 + 85 * k, :],
                      preferred_element_type=jnp.float32)
  for k in range(4):
    out_ref[k] = hw[:, 128 * k:128 * (k + 1)]


def _make_tc_mid(K_in, Fc_in, K_out, Fc_out):
  def body(agg_ref, hw_ref, dinv_ref, b_ref, w_ref, out_ref):
    aggb = jnp.concatenate([agg_ref[k] for k in range(K_in)], axis=1)
    hwb = jnp.concatenate([hw_ref[k] for k in range(K_in)], axis=1)
    dv = dinv_ref[...]
    pre = aggb + dv * dv * hwb + b_ref[...]
    h = jnp.where(pre >= 0, pre, LRELU * pre)
    hw = jnp.dot(h, w_ref[...], preferred_element_type=jnp.float32)
    for k in range(K_out):
      out_ref[k] = hw[:, Fc_out * k:Fc_out * (k + 1)]
  return body


_tc_mid2_body = _make_tc_mid(4, 128, 2, 128)


def _tc_mid3_body(agg_ref, hw_ref, dinv_ref, b_ref, w_ref, out_ref):
  aggb = jnp.concatenate([agg_ref[k] for k in range(2)], axis=1)
  hwb = jnp.concatenate([hw_ref[k] for k in range(2)], axis=1)
  dv = dinv_ref[...]
  pre = aggb + dv * dv * hwb + b_ref[...]
  h = jnp.where(pre >= 0, pre, LRELU * pre)
  hw = jnp.dot(h, w_ref[...], preferred_element_type=jnp.float32)
  out_ref[...] = jnp.concatenate(
      [hw, jnp.zeros((BN, 64), jnp.float32)], axis=1)


def _tc_final_body(agg_ref, hw_ref, dinv_ref, b_ref, wp_ref, bp_ref, out_ref):
  aggb = (agg_ref[0] + agg_ref[1])[:, 0:64]
  hwb = hw_ref[...][:, 0:64]
  dv = dinv_ref[...]
  pre = aggb + dv * dv * hwb + b_ref[...]
  h = jnp.where(pre >= 0, pre, LRELU * pre)
  out_ref[...] = jnp.dot(h, wp_ref[...],
                         preferred_element_type=jnp.float32) + bp_ref[...]


def _row_blk(shape):
  return pl.BlockSpec((BN,) + shape[1:], lambda i: (i,) + (0,) * (len(shape) - 1))


def _chunk_blk(K, Fc):
  return pl.BlockSpec((K, BN, Fc), lambda i: (0, i, 0))


def _full_blk(shape):
  return pl.BlockSpec(shape, lambda i: (0,) * len(shape))


def kernel(x, edge_index, edge_attr, layer_emb, color_emb, relsize_emb,
           W1, b1, W2, b2, W3, b3, Wp, bp):
  src = edge_index[0]
  dst = edge_index[1]
  grid = (N // BN,)

  deg_parts = _sc_deg(dst, edge_attr)
  dinv = pl.pallas_call(
      _tc_dinv_body,
      grid=grid,
      in_specs=[_row_blk((N, NW))],
      out_specs=_row_blk((N, 1)),
      out_shape=jax.ShapeDtypeStruct((N, 1), jnp.float32),
  )(deg_parts.T)
  dinv_flat = dinv.reshape(N)

  hw1 = pl.pallas_call(
      _tc_prep_body,
      grid=grid,
      in_specs=[
          _row_blk(x.shape),
          _full_blk((3, 250)),
          _full_blk((11, 250)),
          _full_blk((256, 85)),
          _full_blk((250 + FS + 250 + 255, 512)),
      ],
      out_specs=_chunk_blk(4, 128),
      out_shape=jax.ShapeDtypeStruct((4, N, 128), jnp.float32),
  )(x, layer_emb, relsize_emb, color_emb, W1)

  agg1 = _sc_edge1(hw1.reshape(4 * N, 128), dinv_flat, src, dst, edge_attr)

  hw2 = pl.pallas_call(
      _tc_mid2_body,
      grid=grid,
      in_specs=[
          _chunk_blk(4, 128),
          _chunk_blk(4, 128),
          _row_blk((N, 1)),
          _full_blk((1, 512)),
          _full_blk((512, 256)),
      ],
      out_specs=_chunk_blk(2, 128),
      out_shape=jax.ShapeDtypeStruct((2, N, 128), jnp.float32),
  )(agg1, hw1, dinv, b1.reshape(1, 512), W2)

  agg2 = _sc_edge2(hw2.reshape(2 * N, 128), dinv_flat, src, dst, edge_attr)

  hw3 = pl.pallas_call(
      _tc_mid3_body,
      grid=grid,
      in_specs=[
          _chunk_blk(2, 128),
          _chunk_blk(2, 128),
          _row_blk((N, 1)),
          _full_blk((1, 256)),
          _full_blk((256, 64)),
      ],
      out_specs=_row_blk((N, 128)),
      out_shape=jax.ShapeDtypeStruct((N, 128), jnp.float32),
  )(agg2, hw2, dinv, b2.reshape(1, 256), W3)

  agg3 = _sc_edge3(hw3, dinv_flat, src, dst, edge_attr)

  out = pl.pallas_call(
      _tc_final_body,
      grid=grid,
      in_specs=[
          _chunk_blk(2, 128),
          _row_blk((N, 128)),
          _row_blk((N, 1)),
          _full_blk((1, 64)),
          _full_blk((64, 3)),
          _full_blk((1, 3)),
      ],
      out_specs=_row_blk((N, 3)),
      out_shape=jax.ShapeDtypeStruct((N, 3), jnp.float32),
  )(agg3, hw3, dinv, b3.reshape(1, 64), Wp, bp.reshape(1, 3))
  return out


# speedup vs baseline: 11.0323x; 1.1165x over previous
import functools

import jax
import jax.numpy as jnp
from jax import lax
from jax.experimental import pallas as pl
from jax.experimental.pallas import tpu as pltpu
from jax.experimental.pallas import tpu_sc as plsc

N = 10000
E = 160000
FS = 1000
NC, NS = 2, 16
NW = NC * NS
NP = 10240
BN = 1000
LRELU = 0.01


def _wrap_clip(idx, n):
  idx = jnp.where(idx < 0, idx + n, idx)
  return jnp.clip(idx, 0, n - 1)


_EPD = E // NW
_DCH = 1000


@functools.partial(
    pl.kernel,
    out_type=jax.ShapeDtypeStruct((NW, N), jnp.float32),
    mesh=plsc.VectorSubcoreMesh(core_axis_name="c", subcore_axis_name="s"),
    scratch_types=[
        pltpu.VMEM((N,), jnp.float32),
        pltpu.VMEM((_DCH + 16,), jnp.int32),
        pltpu.VMEM((_DCH + 16,), jnp.float32),
    ],
    compiler_params=pltpu.CompilerParams(needs_layout_passes=False),
    name="sc_deg",
)
def _sc_deg(dst_hbm, ew_hbm, out_hbm, deg_v, idx_v, w_v):
  c = lax.axis_index("c")
  s = lax.axis_index("s")
  wid = s * NC + c

  def zero(i, _):
    deg_v[pl.ds(i * 16, 16)] = jnp.zeros((16,), jnp.float32)
    return 0

  lax.fori_loop(0, N // 16, zero, 0)
  idx_v[pl.ds(_DCH, 16)] = jnp.zeros((16,), jnp.int32)
  w_v[pl.ds(_DCH, 16)] = jnp.zeros((16,), jnp.float32)
  base0 = wid * _EPD
  ngroups = (_DCH + 15) // 16

  def chunk(ci, _):
    b = pl.multiple_of(base0 + ci * _DCH, 8)
    pltpu.sync_copy(dst_hbm.at[pl.ds(b, _DCH)], idx_v.at[pl.ds(0, _DCH)])
    pltpu.sync_copy(ew_hbm.at[pl.ds(b, _DCH)], w_v.at[pl.ds(0, _DCH)])

    def acc(g, _):
      sl = pl.ds(g * 16, 16)
      plsc.addupdate_scatter(deg_v, [idx_v[sl]], w_v[sl])
      return 0

    lax.fori_loop(0, ngroups, acc, 0)
    return 0

  lax.fori_loop(0, _EPD // _DCH, chunk, 0)
  pltpu.sync_copy(deg_v, out_hbm.at[wid])


@functools.partial(
    pl.kernel,
    out_type=jax.ShapeDtypeStruct((E,), jnp.float32),
    mesh=plsc.VectorSubcoreMesh(core_axis_name="c", subcore_axis_name="s"),
    scratch_types=[
        pltpu.VMEM((N,), jnp.float32),
        pltpu.VMEM((_DCH + 16,), jnp.int32),
        pltpu.VMEM((_DCH + 16,), jnp.int32),
        pltpu.VMEM((_DCH + 16,), jnp.float32),
        pltpu.VMEM((_DCH + 16,), jnp.float32),
    ],
    compiler_params=pltpu.CompilerParams(needs_layout_passes=False),
    name="sc_factors",
)
def _sc_factors(dinv_hbm, src_hbm, dst_hbm, ew_hbm, out_hbm,
                dinv_v, s_v, d_v, w_v, f_v):
  c = lax.axis_index("c")
  s = lax.axis_index("s")
  wid = s * NC + c
  pltpu.sync_copy(dinv_hbm, dinv_v)
  s_v[pl.ds(_DCH, 16)] = jnp.zeros((16,), jnp.int32)
  d_v[pl.ds(_DCH, 16)] = jnp.zeros((16,), jnp.int32)
  base0 = wid * _EPD
  ngroups = (_DCH + 15) // 16

  def chunk(ci, _):
    b = pl.multiple_of(base0 + ci * _DCH, 8)
    pltpu.sync_copy(src_hbm.at[pl.ds(b, _DCH)], s_v.at[pl.ds(0, _DCH)])
    pltpu.sync_copy(dst_hbm.at[pl.ds(b, _DCH)], d_v.at[pl.ds(0, _DCH)])
    pltpu.sync_copy(ew_hbm.at[pl.ds(b, _DCH)], w_v.at[pl.ds(0, _DCH)])

    def grp(g, _):
      sl = pl.ds(g * 16, 16)
      dv_s = plsc.load_gather(dinv_v, [s_v[sl]])
      dv_d = plsc.load_gather(dinv_v, [d_v[sl]])
      f_v[sl] = dv_s * dv_d * w_v[sl]
      return 0

    lax.fori_loop(0, ngroups, grp, 0)
    pltpu.sync_copy(f_v.at[pl.ds(0, _DCH)], out_hbm.at[pl.ds(b, _DCH)])
    return 0

  lax.fori_loop(0, _EPD // _DCH, chunk, 0)


def _make_sc_edge(K, name):
  Fc = 128
  CPS = max(1, K // NC)
  EP = E // NS if K >= 2 else E // NW
  B = 80 if K >= 2 else 40
  BP = B if B % 16 == 0 else B + 16 - B % 16
  NB = EP // B
  SB = 25
  SE = SB * B
  SP = SE if SE % 16 == 0 else SE + 16 - SE % 16
  RPT = NP // NS
  ZR = 16
  n_out = K if K >= 2 else NC
  T_STEADY = (NB - 5) // 3

  @functools.partial(
      pl.kernel,
      out_type=jax.ShapeDtypeStruct((n_out, NP, Fc), jnp.float32),
      mesh=plsc.VectorSubcoreMesh(core_axis_name="c", subcore_axis_name="s"),
      scratch_types=[
          pltpu.VMEM((SP,), jnp.int32),
          pltpu.VMEM((SP,), jnp.int32),
          pltpu.VMEM((SP,), jnp.float32),
          pltpu.VMEM((BP,), jnp.int32),
          pltpu.VMEM((BP,), jnp.int32),
          pltpu.VMEM((BP,), jnp.int32),
          pltpu.VMEM((BP,), jnp.int32),
          pltpu.VMEM((BP,), jnp.int32),
          pltpu.VMEM((BP,), jnp.int32),
          pltpu.VMEM((BP,), jnp.float32),
          pltpu.VMEM((BP,), jnp.float32),
          pltpu.VMEM((BP,), jnp.float32),
          pltpu.VMEM((BP, Fc), jnp.float32),
          pltpu.VMEM((BP, Fc), jnp.float32),
          pltpu.VMEM((BP, Fc), jnp.float32),
          pltpu.VMEM((ZR, Fc), jnp.float32),
          pltpu.VMEM_SHARED((NP, Fc), jnp.float32),
          pltpu.SemaphoreType.DMA,
          pltpu.SemaphoreType.DMA,
          pltpu.SemaphoreType.DMA,
          pltpu.SemaphoreType.DMA,
          pltpu.SemaphoreType.DMA,
          pltpu.SemaphoreType.DMA,
      ],
      compiler_params=pltpu.CompilerParams(needs_layout_passes=False),
      name=name,
  )
  def edge(hw_hbm, f_hbm, src_hbm, dst_hbm, out_hbm,
           sa_v, da_v, fa_v, si0, si1, si2, di0, di1, di2, f0, f1, f2,
           rows0, rows1, rows2, zero_v, agg_sh,
           gs0, gs1, gs2, ss0, ss1, ss2):
    c = lax.axis_index("c")
    s = lax.axis_index("s")
    slots = [(si0, di0, f0, rows0, gs0, ss0),
             (si1, di1, f1, rows1, gs1, ss1),
             (si2, di2, f2, rows2, gs2, ss2)]
    if SP != SE:
      sa_v[pl.ds(SP - 16, 16)] = jnp.zeros((16,), jnp.int32)
      da_v[pl.ds(SP - 16, 16)] = jnp.zeros((16,), jnp.int32)
      wpad = pl.ds(SP - 16, 16)
      fa_v[wpad] = jnp.zeros((16,), jnp.float32)
    if K >= 2:
      edge_base = s * EP
    else:
      edge_base = c * (E // NC) + s * EP
    edge_base = pl.multiple_of(edge_base, 8)

    def zz(i, _):
      for j in range(Fc // 16):
        zero_v[i, pl.ds(j * 16, 16)] = jnp.zeros((16,), jnp.float32)
      return 0

    lax.fori_loop(0, ZR, zz, 0)
    iota16 = lax.iota(jnp.int32, 16)
    padmask = iota16 < (16 - (BP - B))
    padf = jnp.where(padmask, 1.0, 0.0)

    def refill(stg):
      sb = pl.multiple_of(edge_base + stg * SE, 8)
      pltpu.sync_copy(src_hbm.at[pl.ds(sb, SE)], sa_v.at[pl.ds(0, SE)])
      pltpu.sync_copy(dst_hbm.at[pl.ds(sb, SE)], da_v.at[pl.ds(0, SE)])
      pltpu.sync_copy(f_hbm.at[pl.ds(sb, SE)], fa_v.at[pl.ds(0, SE)])

    def scale(f_x, rows_x):
      def grp(g, _):
        fvec = f_x[pl.ds(g * 16, 16)]
        for lane in range(16):
          i = g * 16 + lane
          fs = fvec[lane]
          for j2 in range(Fc // 16):
            sl2 = pl.ds(j2 * 16, 16)
            rows_x[i, sl2] = rows_x[i, sl2] * fs
        return 0
      lax.fori_loop(0, BP // 16, grp, 0)

    for cc in range(CPS):
      kk = c + cc * NC if K >= 2 else 0

      def fstage(lb, si_x, di_x, f_x):
        off = lb * B
        for g in range(BP // 16):
          sl = pl.ds(g * 16, 16)
          so = pl.ds(off + g * 16, 16)
          f_x[sl] = fa_v[so]
          di_x[sl] = da_v[so]
          if K >= 2:
            si_x[sl] = sa_v[so] + kk * N
          else:
            si_x[sl] = sa_v[so]
        if BP != B:
          tl = pl.ds(BP - 16, 16)
          f_x[tl] = f_x[tl] * padf
          di_x[tl] = jnp.where(padmask, di_x[tl], N + s * 8 + (iota16 - 8))

      for z in range(RPT // ZR):
        pltpu.sync_copy(zero_v, agg_sh.at[pl.ds(s * RPT + z * ZR, ZR)])
      plsc.subcore_barrier()

      refill(0)
      for j in range(3):
        sx, dx, fx, rx, gs, _ = slots[j]
        fstage(j, sx, dx, fx)
        pltpu.async_copy(hw_hbm.at[sx], rx, gs)

      def steady(t, _):
        b0 = t * 3
        for j in range(3):
          sx, dx, fx, rx, gs, ss = slots[j]
          pltpu.make_async_copy(hw_hbm.at[sx], rx, gs).wait()
          scale(fx, rx)
          pltpu.async_copy(rx, agg_sh.at[dx], ss, add=True)
        for j in range(3):
          sx, dx, fx, rx, gs, ss = slots[j]
          pltpu.make_async_copy(rx, agg_sh.at[dx], ss).wait()
          bn = b0 + 3 + j
          stg = bn // SB
          lb = bn - stg * SB

          @pl.when(lb == 0)
          def _():
            refill(stg)

          fstage(lb, sx, dx, fx)
          pltpu.async_copy(hw_hbm.at[sx], rx, gs)
        return 0

      lax.fori_loop(0, T_STEADY, steady, 0)
      for j in range(3):
        sx, dx, fx, rx, gs, ss = slots[j]
        pltpu.make_async_copy(hw_hbm.at[sx], rx, gs).wait()
        scale(fx, rx)
        pltpu.sync_copy(rx, agg_sh.at[dx], add=True)
      for j in range(NB - 3 * T_STEADY - 3):
        bn = 3 * T_STEADY + 3 + j
        sx, dx, fx, rx, gs, ss = slots[j]
        fstage(bn % SB, sx, dx, fx)
        pltpu.async_copy(hw_hbm.at[sx], rx, gs).wait()
        scale(fx, rx)
        pltpu.sync_copy(rx, agg_sh.at[dx], add=True)

      plsc.subcore_barrier()
      out_idx = kk if K >= 2 else c
      pltpu.sync_copy(agg_sh.at[pl.ds(s * RPT, RPT)],
                      out_hbm.at[out_idx, pl.ds(s * RPT, RPT)])
      if cc + 1 < CPS:
        plsc.subcore_barrier()

  return edge


_sc_edge1 = _make_sc_edge(4, "sc_edge1")
_sc_edge2 = _make_sc_edge(2, "sc_edge2")
_sc_edge3 = _make_sc_edge(1, "sc_edge3")


def _tc_dinv_body(dp_ref, out_ref):
  deg = jnp.sum(dp_ref[...], axis=1, keepdims=True) + 1.0
  out_ref[...] = jnp.where(deg > 0, lax.rsqrt(deg), 0.0)


def _tc_prep_body(x_ref, lemb_ref, remb_ref, cemb_ref, w1_ref, out_ref):
  xb = x_ref[...]
  lid = _wrap_clip((xb[:, 0:1] - 1.0).astype(jnp.int32), 3)
  rid = _wrap_clip(
      jnp.round(jnp.abs(xb[:, FS + 1:FS + 2]) * 10.0).astype(jnp.int32), 11)
  resnet = xb[:, 1:1 + FS]

  hw = jnp.dot(resnet, w1_ref[250:250 + FS, :],
               preferred_element_type=jnp.float32)
  for t in range(3):
    row = jnp.dot(lemb_ref[t:t + 1, :], w1_ref[0:250, :],
                  preferred_element_type=jnp.float32)
    hw = hw + jnp.where(lid == t, 1.0, 0.0) * row
  for t in range(11):
    row = jnp.dot(remb_ref[t:t + 1, :], w1_ref[1250:1500, :],
                  preferred_element_type=jnp.float32)
    hw = hw + jnp.where(rid == t, 1.0, 0.0) * row
  iot = lax.broadcasted_iota(jnp.int32, (BN, 256), 1)
  for k in range(3):
    cid = _wrap_clip(xb[:, FS + 2 + k:FS + 3 + k].astype(jnp.int32), 256)
    oh = (iot == cid).astype(jnp.float32)
    col = jnp.dot(oh, cemb_ref[...], preferred_element_type=jnp.float32)
    hw = hw + jnp.dot(col, w1_ref[1500 + 85 * k:1585 + 85 * k, :],
                      preferred_element_type=jnp.float32)
  for k in range(4):
    out_ref[k] = hw[:, 128 * k:128 * (k + 1)]


def _make_tc_mid(K_in, Fc_in, K_out, Fc_out):
  def body(agg_ref, hw_ref, dinv_ref, b_ref, w_ref, out_ref):
    aggb = jnp.concatenate([agg_ref[k] for k in range(K_in)], axis=1)
    hwb = jnp.concatenate([hw_ref[k] for k in range(K_in)], axis=1)
    dv = dinv_ref[...]
    pre = aggb + dv * dv * hwb + b_ref[...]
    h = jnp.where(pre >= 0, pre, LRELU * pre)
    hw = jnp.dot(h, w_ref[...], preferred_element_type=jnp.float32)
    for k in range(K_out):
      out_ref[k] = hw[:, Fc_out * k:Fc_out * (k + 1)]
  return body


_tc_mid2_body = _make_tc_mid(4, 128, 2, 128)


def _tc_mid3_body(agg_ref, hw_ref, dinv_ref, b_ref, w_ref, out_ref):
  aggb = jnp.concatenate([agg_ref[k] for k in range(2)], axis=1)
  hwb = jnp.concatenate([hw_ref[k] for k in range(2)], axis=1)
  dv = dinv_ref[...]
  pre = aggb + dv * dv * hwb + b_ref[...]
  h = jnp.where(pre >= 0, pre, LRELU * pre)
  hw = jnp.dot(h, w_ref[...], preferred_element_type=jnp.float32)
  out_ref[...] = jnp.concatenate(
      [hw, jnp.zeros((BN, 64), jnp.float32)], axis=1)


def _tc_final_body(agg_ref, hw_ref, dinv_ref, b_ref, wp_ref, bp_ref, out_ref):
  aggb = (agg_ref[0] + agg_ref[1])[:, 0:64]
  hwb = hw_ref[...][:, 0:64]
  dv = dinv_ref[...]
  pre = aggb + dv * dv * hwb + b_ref[...]
  h = jnp.where(pre >= 0, pre, LRELU * pre)
  out_ref[...] = jnp.dot(h, wp_ref[...],
                         preferred_element_type=jnp.float32) + bp_ref[...]


def _row_blk(shape):
  return pl.BlockSpec((BN,) + shape[1:], lambda i: (i,) + (0,) * (len(shape) - 1))


def _chunk_blk(K, Fc):
  return pl.BlockSpec((K, BN, Fc), lambda i: (0, i, 0))


def _full_blk(shape):
  return pl.BlockSpec(shape, lambda i: (0,) * len(shape))


def kernel(x, edge_index, edge_attr, layer_emb, color_emb, relsize_emb,
           W1, b1, W2, b2, W3, b3, Wp, bp):
  src = edge_index[0]
  dst = edge_index[1]
  grid = (N // BN,)

  deg_parts = _sc_deg(dst, edge_attr)
  dinv = pl.pallas_call(
      _tc_dinv_body,
      grid=grid,
      in_specs=[_row_blk((N, NW))],
      out_specs=_row_blk((N, 1)),
      out_shape=jax.ShapeDtypeStruct((N, 1), jnp.float32),
  )(deg_parts.T)
  dinv_flat = dinv.reshape(N)
  fedge = _sc_factors(dinv_flat, src, dst, edge_attr)

  hw1 = pl.pallas_call(
      _tc_prep_body,
      grid=grid,
      in_specs=[
          _row_blk(x.shape),
          _full_blk((3, 250)),
          _full_blk((11, 250)),
          _full_blk((256, 85)),
          _full_blk((250 + FS + 250 + 255, 512)),
      ],
      out_specs=_chunk_blk(4, 128),
      out_shape=jax.ShapeDtypeStruct((4, N, 128), jnp.float32),
  )(x, layer_emb, relsize_emb, color_emb, W1)

  agg1 = _sc_edge1(hw1.reshape(4 * N, 128), fedge, src, dst)

  hw2 = pl.pallas_call(
      _tc_mid2_body,
      grid=grid,
      in_specs=[
          _chunk_blk(4, 128),
          _chunk_blk(4, 128),
          _row_blk((N, 1)),
          _full_blk((1, 512)),
          _full_blk((512, 256)),
      ],
      out_specs=_chunk_blk(2, 128),
      out_shape=jax.ShapeDtypeStruct((2, N, 128), jnp.float32),
  )(agg1, hw1, dinv, b1.reshape(1, 512), W2)

  agg2 = _sc_edge2(hw2.reshape(2 * N, 128), fedge, src, dst)

  hw3 = pl.pallas_call(
      _tc_mid3_body,
      grid=grid,
      in_specs=[
          _chunk_blk(2, 128),
          _chunk_blk(2, 128),
          _row_blk((N, 1)),
          _full_blk((1, 256)),
          _full_blk((256, 64)),
      ],
      out_specs=_row_blk((N, 128)),
      out_shape=jax.ShapeDtypeStruct((N, 128), jnp.float32),
  )(agg2, hw2, dinv, b2.reshape(1, 256), W3)

  agg3 = _sc_edge3(hw3, fedge, src, dst)

  out = pl.pallas_call(
      _tc_final_body,
      grid=grid,
      in_specs=[
          _chunk_blk(2, 128),
          _row_blk((N, 128)),
          _row_blk((N, 1)),
          _full_blk((1, 64)),
          _full_blk((64, 3)),
          _full_blk((1, 3)),
      ],
      out_specs=_row_blk((N, 3)),
      out_shape=jax.ShapeDtypeStruct((N, 3), jnp.float32),
  )(agg3, hw3, dinv, b3.reshape(1, 64), Wp, bp.reshape(1, 3))
  return out


# speedup vs baseline: 12.4135x; 1.1252x over previous
import functools

import jax
import jax.numpy as jnp
from jax import lax
from jax.experimental import pallas as pl
from jax.experimental.pallas import tpu as pltpu
from jax.experimental.pallas import tpu_sc as plsc

N = 10000
E = 160000
FS = 1000
NC, NS = 2, 16
NW = NC * NS
NP = 10240
BN = 1000
LRELU = 0.01


def _wrap_clip(idx, n):
  idx = jnp.where(idx < 0, idx + n, idx)
  return jnp.clip(idx, 0, n - 1)


_EPD = E // NW
_DCH = 1000


@functools.partial(
    pl.kernel,
    out_type=jax.ShapeDtypeStruct((NW, N), jnp.float32),
    mesh=plsc.VectorSubcoreMesh(core_axis_name="c", subcore_axis_name="s"),
    scratch_types=[
        pltpu.VMEM((N,), jnp.float32),
        pltpu.VMEM((_DCH + 16,), jnp.int32),
        pltpu.VMEM((_DCH + 16,), jnp.float32),
    ],
    compiler_params=pltpu.CompilerParams(needs_layout_passes=False),
    name="sc_deg",
)
def _sc_deg(dst_hbm, ew_hbm, out_hbm, deg_v, idx_v, w_v):
  c = lax.axis_index("c")
  s = lax.axis_index("s")
  wid = s * NC + c

  def zero(i, _):
    deg_v[pl.ds(i * 16, 16)] = jnp.zeros((16,), jnp.float32)
    return 0

  lax.fori_loop(0, N // 16, zero, 0)
  idx_v[pl.ds(_DCH, 16)] = jnp.zeros((16,), jnp.int32)
  w_v[pl.ds(_DCH, 16)] = jnp.zeros((16,), jnp.float32)
  base0 = wid * _EPD
  ngroups = (_DCH + 15) // 16

  def chunk(ci, _):
    b = pl.multiple_of(base0 + ci * _DCH, 8)
    pltpu.sync_copy(dst_hbm.at[pl.ds(b, _DCH)], idx_v.at[pl.ds(0, _DCH)])
    pltpu.sync_copy(ew_hbm.at[pl.ds(b, _DCH)], w_v.at[pl.ds(0, _DCH)])

    def acc(g, _):
      sl = pl.ds(g * 16, 16)
      plsc.addupdate_scatter(deg_v, [idx_v[sl]], w_v[sl])
      return 0

    lax.fori_loop(0, ngroups, acc, 0)
    return 0

  lax.fori_loop(0, _EPD // _DCH, chunk, 0)
  pltpu.sync_copy(deg_v, out_hbm.at[wid])


@functools.partial(
    pl.kernel,
    out_type=jax.ShapeDtypeStruct((E,), jnp.float32),
    mesh=plsc.VectorSubcoreMesh(core_axis_name="c", subcore_axis_name="s"),
    scratch_types=[
        pltpu.VMEM((N,), jnp.float32),
        pltpu.VMEM((_DCH + 16,), jnp.int32),
        pltpu.VMEM((_DCH + 16,), jnp.int32),
        pltpu.VMEM((_DCH + 16,), jnp.float32),
        pltpu.VMEM((_DCH + 16,), jnp.float32),
    ],
    compiler_params=pltpu.CompilerParams(needs_layout_passes=False),
    name="sc_factors",
)
def _sc_factors(dinv_hbm, src_hbm, dst_hbm, ew_hbm, out_hbm,
                dinv_v, s_v, d_v, w_v, f_v):
  c = lax.axis_index("c")
  s = lax.axis_index("s")
  wid = s * NC + c
  pltpu.sync_copy(dinv_hbm, dinv_v)
  s_v[pl.ds(_DCH, 16)] = jnp.zeros((16,), jnp.int32)
  d_v[pl.ds(_DCH, 16)] = jnp.zeros((16,), jnp.int32)
  base0 = wid * _EPD
  ngroups = (_DCH + 15) // 16

  def chunk(ci, _):
    b = pl.multiple_of(base0 + ci * _DCH, 8)
    pltpu.sync_copy(src_hbm.at[pl.ds(b, _DCH)], s_v.at[pl.ds(0, _DCH)])
    pltpu.sync_copy(dst_hbm.at[pl.ds(b, _DCH)], d_v.at[pl.ds(0, _DCH)])
    pltpu.sync_copy(ew_hbm.at[pl.ds(b, _DCH)], w_v.at[pl.ds(0, _DCH)])

    def grp(g, _):
      sl = pl.ds(g * 16, 16)
      dv_s = plsc.load_gather(dinv_v, [s_v[sl]])
      dv_d = plsc.load_gather(dinv_v, [d_v[sl]])
      f_v[sl] = dv_s * dv_d * w_v[sl]
      return 0

    lax.fori_loop(0, ngroups, grp, 0)
    pltpu.sync_copy(f_v.at[pl.ds(0, _DCH)], out_hbm.at[pl.ds(b, _DCH)])
    return 0

  lax.fori_loop(0, _EPD // _DCH, chunk, 0)


def _make_sc_edge(K, name):
  Fc = 128
  CPS = max(1, K // NC)
  EP = E // NS if K >= 2 else E // NW
  B = 80 if K >= 2 else 40
  BP = B if B % 16 == 0 else B + 16 - B % 16
  NB = EP // B
  SB = 25
  SE = SB * B
  SP = SE if SE % 16 == 0 else SE + 16 - SE % 16
  RPT = NP // NS
  ZR = 16
  n_out = K if K >= 2 else NC
  T_STEADY = (NB - 5) // 3

  @functools.partial(
      pl.kernel,
      out_type=jax.ShapeDtypeStruct((n_out, NP, Fc), jnp.float32),
      mesh=plsc.VectorSubcoreMesh(core_axis_name="c", subcore_axis_name="s"),
      scratch_types=[
          pltpu.VMEM((SP,), jnp.int32),
          pltpu.VMEM((SP,), jnp.int32),
          pltpu.VMEM((SP,), jnp.float32),
          pltpu.VMEM((BP,), jnp.int32),
          pltpu.VMEM((BP,), jnp.int32),
          pltpu.VMEM((BP,), jnp.int32),
          pltpu.VMEM((BP,), jnp.int32),
          pltpu.VMEM((BP,), jnp.int32),
          pltpu.VMEM((BP,), jnp.int32),
          pltpu.VMEM((BP,), jnp.float32),
          pltpu.VMEM((BP,), jnp.float32),
          pltpu.VMEM((BP,), jnp.float32),
          pltpu.VMEM((BP, Fc), jnp.float32),
          pltpu.VMEM((BP, Fc), jnp.float32),
          pltpu.VMEM((BP, Fc), jnp.float32),
          pltpu.VMEM((ZR, Fc), jnp.float32),
          pltpu.VMEM_SHARED((NP, Fc), jnp.float32),
          pltpu.SemaphoreType.DMA,
          pltpu.SemaphoreType.DMA,
          pltpu.SemaphoreType.DMA,
          pltpu.SemaphoreType.DMA,
          pltpu.SemaphoreType.DMA,
          pltpu.SemaphoreType.DMA,
      ],
      compiler_params=pltpu.CompilerParams(needs_layout_passes=False),
      name=name,
  )
  def edge(hw_hbm, f_hbm, src_hbm, dst_hbm, out_hbm,
           sa_v, da_v, fa_v, si0, si1, si2, di0, di1, di2, f0, f1, f2,
           rows0, rows1, rows2, zero_v, agg_sh,
           gs0, gs1, gs2, ss0, ss1, ss2):
    c = lax.axis_index("c")
    s = lax.axis_index("s")
    slots = [(si0, di0, f0, rows0, gs0, ss0),
             (si1, di1, f1, rows1, gs1, ss1),
             (si2, di2, f2, rows2, gs2, ss2)]
    if SP != SE:
      sa_v[pl.ds(SP - 16, 16)] = jnp.zeros((16,), jnp.int32)
      da_v[pl.ds(SP - 16, 16)] = jnp.zeros((16,), jnp.int32)
      wpad = pl.ds(SP - 16, 16)
      fa_v[wpad] = jnp.zeros((16,), jnp.float32)
    if K >= 2:
      edge_base = s * EP
    else:
      edge_base = c * (E // NC) + s * EP
    edge_base = pl.multiple_of(edge_base, 8)

    def zz(i, _):
      for j in range(Fc // 16):
        zero_v[i, pl.ds(j * 16, 16)] = jnp.zeros((16,), jnp.float32)
      return 0

    lax.fori_loop(0, ZR, zz, 0)
    iota16 = lax.iota(jnp.int32, 16)
    padmask = iota16 < (16 - (BP - B))
    padf = jnp.where(padmask, 1.0, 0.0)

    def refill(stg):
      sb = pl.multiple_of(edge_base + stg * SE, 8)
      pltpu.sync_copy(src_hbm.at[pl.ds(sb, SE)], sa_v.at[pl.ds(0, SE)])
      pltpu.sync_copy(dst_hbm.at[pl.ds(sb, SE)], da_v.at[pl.ds(0, SE)])
      pltpu.sync_copy(f_hbm.at[pl.ds(sb, SE)], fa_v.at[pl.ds(0, SE)])

    def scale(f_x, rows_x):
      @functools.partial(plsc.parallel_loop, 0, BP // 16, unroll=2)
      def _grp(g):
        fvec = f_x[pl.ds(g * 16, 16)]
        for lane in range(16):
          i = g * 16 + lane
          fs = fvec[lane]
          for j2 in range(Fc // 16):
            sl2 = pl.ds(j2 * 16, 16)
            rows_x[i, sl2] = rows_x[i, sl2] * fs

    for cc in range(CPS):
      kk = c + cc * NC if K >= 2 else 0

      def fstage(lb, si_x, di_x, f_x):
        off = lb * B
        for g in range(BP // 16):
          sl = pl.ds(g * 16, 16)
          so = pl.ds(off + g * 16, 16)
          f_x[sl] = fa_v[so]
          di_x[sl] = da_v[so]
          if K >= 2:
            si_x[sl] = sa_v[so] + kk * N
          else:
            si_x[sl] = sa_v[so]
        if BP != B:
          tl = pl.ds(BP - 16, 16)
          f_x[tl] = f_x[tl] * padf
          di_x[tl] = jnp.where(padmask, di_x[tl], N + s * 8 + (iota16 - 8))

      for z in range(RPT // ZR):
        pltpu.sync_copy(zero_v, agg_sh.at[pl.ds(s * RPT + z * ZR, ZR)])
      plsc.subcore_barrier()

      refill(0)
      for j in range(3):
        sx, dx, fx, rx, gs, _ = slots[j]
        fstage(j, sx, dx, fx)
        pltpu.async_copy(hw_hbm.at[sx], rx, gs)

      def steady(t, _):
        b0 = t * 3
        for j in range(3):
          sx, dx, fx, rx, gs, ss = slots[j]
          pltpu.make_async_copy(hw_hbm.at[sx], rx, gs).wait()
          scale(fx, rx)
          pltpu.async_copy(rx, agg_sh.at[dx], ss, add=True)
        for j in range(3):
          sx, dx, fx, rx, gs, ss = slots[j]
          pltpu.make_async_copy(rx, agg_sh.at[dx], ss).wait()
          bn = b0 + 3 + j
          stg = bn // SB
          lb = bn - stg * SB

          @pl.when(lb == 0)
          def _():
            refill(stg)

          fstage(lb, sx, dx, fx)
          pltpu.async_copy(hw_hbm.at[sx], rx, gs)
        return 0

      lax.fori_loop(0, T_STEADY, steady, 0)
      for j in range(3):
        sx, dx, fx, rx, gs, ss = slots[j]
        pltpu.make_async_copy(hw_hbm.at[sx], rx, gs).wait()
        scale(fx, rx)
        pltpu.sync_copy(rx, agg_sh.at[dx], add=True)
      for j in range(NB - 3 * T_STEADY - 3):
        bn = 3 * T_STEADY + 3 + j
        sx, dx, fx, rx, gs, ss = slots[j]
        fstage(bn % SB, sx, dx, fx)
        pltpu.async_copy(hw_hbm.at[sx], rx, gs).wait()
        scale(fx, rx)
        pltpu.sync_copy(rx, agg_sh.at[dx], add=True)

      plsc.subcore_barrier()
      out_idx = kk if K >= 2 else c
      pltpu.sync_copy(agg_sh.at[pl.ds(s * RPT, RPT)],
                      out_hbm.at[out_idx, pl.ds(s * RPT, RPT)])
      if cc + 1 < CPS:
        plsc.subcore_barrier()

  return edge


_sc_edge1 = _make_sc_edge(4, "sc_edge1")
_sc_edge2 = _make_sc_edge(2, "sc_edge2")
_sc_edge3 = _make_sc_edge(1, "sc_edge3")


def _tc_dinv_body(dp_ref, out_ref):
  deg = jnp.sum(dp_ref[...], axis=1, keepdims=True) + 1.0
  out_ref[...] = jnp.where(deg > 0, lax.rsqrt(deg), 0.0)


def _tc_prep_body(x_ref, lemb_ref, remb_ref, cemb_ref, w1_ref, out_ref):
  xb = x_ref[...]
  lid = _wrap_clip((xb[:, 0:1] - 1.0).astype(jnp.int32), 3)
  rid = _wrap_clip(
      jnp.round(jnp.abs(xb[:, FS + 1:FS + 2]) * 10.0).astype(jnp.int32), 11)
  resnet = xb[:, 1:1 + FS]

  hw = jnp.dot(resnet, w1_ref[250:250 + FS, :],
               preferred_element_type=jnp.float32)
  for t in range(3):
    row = jnp.dot(lemb_ref[t:t + 1, :], w1_ref[0:250, :],
                  preferred_element_type=jnp.float32)
    hw = hw + jnp.where(lid == t, 1.0, 0.0) * row
  for t in range(11):
    row = jnp.dot(remb_ref[t:t + 1, :], w1_ref[1250:1500, :],
                  preferred_element_type=jnp.float32)
    hw = hw + jnp.where(rid == t, 1.0, 0.0) * row
  iot = lax.broadcasted_iota(jnp.int32, (BN, 256), 1)
  for k in range(3):
    cid = _wrap_clip(xb[:, FS + 2 + k:FS + 3 + k].astype(jnp.int32), 256)
    oh = (iot == cid).astype(jnp.float32)
    col = jnp.dot(oh, cemb_ref[...], preferred_element_type=jnp.float32)
    hw = hw + jnp.dot(col, w1_ref[1500 + 85 * k:1585 + 85 * k, :],
                      preferred_element_type=jnp.float32)
  for k in range(4):
    out_ref[k] = hw[:, 128 * k:128 * (k + 1)]


def _make_tc_mid(K_in, Fc_in, K_out, Fc_out):
  def body(agg_ref, hw_ref, dinv_ref, b_ref, w_ref, out_ref):
    aggb = jnp.concatenate([agg_ref[k] for k in range(K_in)], axis=1)
    hwb = jnp.concatenate([hw_ref[k] for k in range(K_in)], axis=1)
    dv = dinv_ref[...]
    pre = aggb + dv * dv * hwb + b_ref[...]
    h = jnp.where(pre >= 0, pre, LRELU * pre)
    hw = jnp.dot(h, w_ref[...], preferred_element_type=jnp.float32)
    for k in range(K_out):
      out_ref[k] = hw[:, Fc_out * k:Fc_out * (k + 1)]
  return body


_tc_mid2_body = _make_tc_mid(4, 128, 2, 128)


def _tc_mid3_body(agg_ref, hw_ref, dinv_ref, b_ref, w_ref, out_ref):
  aggb = jnp.concatenate([agg_ref[k] for k in range(2)], axis=1)
  hwb = jnp.concatenate([hw_ref[k] for k in range(2)], axis=1)
  dv = dinv_ref[...]
  pre = aggb + dv * dv * hwb + b_ref[...]
  h = jnp.where(pre >= 0, pre, LRELU * pre)
  hw = jnp.dot(h, w_ref[...], preferred_element_type=jnp.float32)
  out_ref[...] = jnp.concatenate(
      [hw, jnp.zeros((BN, 64), jnp.float32)], axis=1)


def _tc_final_body(agg_ref, hw_ref, dinv_ref, b_ref, wp_ref, bp_ref, out_ref):
  aggb = (agg_ref[0] + agg_ref[1])[:, 0:64]
  hwb = hw_ref[...][:, 0:64]
  dv = dinv_ref[...]
  pre = aggb + dv * dv * hwb + b_ref[...]
  h = jnp.where(pre >= 0, pre, LRELU * pre)
  out_ref[...] = jnp.dot(h, wp_ref[...],
                         preferred_element_type=jnp.float32) + bp_ref[...]


def _row_blk(shape):
  return pl.BlockSpec((BN,) + shape[1:], lambda i: (i,) + (0,) * (len(shape) - 1))


def _chunk_blk(K, Fc):
  return pl.BlockSpec((K, BN, Fc), lambda i: (0, i, 0))


def _full_blk(shape):
  return pl.BlockSpec(shape, lambda i: (0,) * len(shape))


def kernel(x, edge_index, edge_attr, layer_emb, color_emb, relsize_emb,
           W1, b1, W2, b2, W3, b3, Wp, bp):
  src = edge_index[0]
  dst = edge_index[1]
  grid = (N // BN,)

  deg_parts = _sc_deg(dst, edge_attr)
  dinv = pl.pallas_call(
      _tc_dinv_body,
      grid=grid,
      in_specs=[_row_blk((N, NW))],
      out_specs=_row_blk((N, 1)),
      out_shape=jax.ShapeDtypeStruct((N, 1), jnp.float32),
  )(deg_parts.T)
  dinv_flat = dinv.reshape(N)
  fedge = _sc_factors(dinv_flat, src, dst, edge_attr)

  hw1 = pl.pallas_call(
      _tc_prep_body,
      grid=grid,
      in_specs=[
          _row_blk(x.shape),
          _full_blk((3, 250)),
          _full_blk((11, 250)),
          _full_blk((256, 85)),
          _full_blk((250 + FS + 250 + 255, 512)),
      ],
      out_specs=_chunk_blk(4, 128),
      out_shape=jax.ShapeDtypeStruct((4, N, 128), jnp.float32),
  )(x, layer_emb, relsize_emb, color_emb, W1)

  agg1 = _sc_edge1(hw1.reshape(4 * N, 128), fedge, src, dst)

  hw2 = pl.pallas_call(
      _tc_mid2_body,
      grid=grid,
      in_specs=[
          _chunk_blk(4, 128),
          _chunk_blk(4, 128),
          _row_blk((N, 1)),
          _full_blk((1, 512)),
          _full_blk((512, 256)),
      ],
      out_specs=_chunk_blk(2, 128),
      out_shape=jax.ShapeDtypeStruct((2, N, 128), jnp.float32),
  )(agg1, hw1, dinv, b1.reshape(1, 512), W2)

  agg2 = _sc_edge2(hw2.reshape(2 * N, 128), fedge, src, dst)

  hw3 = pl.pallas_call(
      _tc_mid3_body,
      grid=grid,
      in_specs=[
          _chunk_blk(2, 128),
          _chunk_blk(2, 128),
          _row_blk((N, 1)),
          _full_blk((1, 256)),
          _full_blk((256, 64)),
      ],
      out_specs=_row_blk((N, 128)),
      out_shape=jax.ShapeDtypeStruct((N, 128), jnp.float32),
  )(agg2, hw2, dinv, b2.reshape(1, 256), W3)

  agg3 = _sc_edge3(hw3, fedge, src, dst)

  out = pl.pallas_call(
      _tc_final_body,
      grid=grid,
      in_specs=[
          _chunk_blk(2, 128),
          _row_blk((N, 128)),
          _row_blk((N, 1)),
          _full_blk((1, 64)),
          _full_blk((64, 3)),
          _full_blk((1, 3)),
      ],
      out_specs=_row_blk((N, 3)),
      out_shape=jax.ShapeDtypeStruct((N, 3), jnp.float32),
  )(agg3, hw3, dinv, b3.reshape(1, 64), Wp, bp.reshape(1, 3))
  return out
